# baseline (speedup 1.0000x reference)
import functools

import jax
import jax.numpy as jnp
from jax import lax
from jax.experimental import pallas as pl
from jax.experimental.pallas import tpu as pltpu
from jax.experimental.pallas import tpu_sc as plsc

CARDS = 100000
EMB_D = 128
CTX = 200
HID = 128

_NC = 2
_NS = 16
_NW = _NC * _NS
_CTX_PAD = 256
_BPW = _CTX_PAD // _NW

_KS = 8
_BK = CTX * EMB_D // _KS

_BV = 8192
_GB = (CARDS + _BV - 1) // _BV


def _sc_gather(table, idx_pad):
    mesh = plsc.VectorSubcoreMesh(core_axis_name="c", subcore_axis_name="s")

    @functools.partial(
        pl.kernel,
        mesh=mesh,
        out_type=jax.ShapeDtypeStruct((_CTX_PAD, EMB_D), jnp.float32),
        scratch_types=[
            pltpu.VMEM((_BPW,), jnp.int32),
            pltpu.VMEM((_BPW, EMB_D), jnp.float32),
            pltpu.SemaphoreType.DMA,
        ],
    )
    def k(table_hbm, idx_hbm, out_hbm, idx_v, rows_v, sem):
        wid = lax.axis_index("s") * _NC + lax.axis_index("c")
        base = wid * _BPW
        pltpu.sync_copy(idx_hbm.at[pl.ds(base, _BPW)], idx_v)
        pltpu.async_copy(table_hbm.at[idx_v], rows_v, sem).wait()
        pltpu.sync_copy(rows_v, out_hbm.at[pl.ds(base, _BPW)])

    return k(table, idx_pad)


def _h_body(x_ref, w1_ref, b1_ref, h_ref, acc_ref):
    j = pl.program_id(0)

    @pl.when(j == 0)
    def _():
        acc_ref[...] = jnp.zeros_like(acc_ref)

    acc_ref[...] += jnp.dot(x_ref[...], w1_ref[...],
                            preferred_element_type=jnp.float32)

    @pl.when(j == _KS - 1)
    def _():
        h_ref[...] = jnp.maximum(acc_ref[...] + b1_ref[...], 0.0)


def _h_layer(x, W1, b1r):
    return pl.pallas_call(
        _h_body,
        grid=(_KS,),
        in_specs=[
            pl.BlockSpec((1, _BK), lambda j: (0, j)),
            pl.BlockSpec((_BK, HID), lambda j: (j, 0)),
            pl.BlockSpec((1, HID), lambda j: (0, 0)),
        ],
        out_specs=pl.BlockSpec((1, HID), lambda j: (0, 0)),
        out_shape=jax.ShapeDtypeStruct((1, HID), jnp.float32),
        scratch_shapes=[pltpu.VMEM((1, HID), jnp.float32)],
    )(x, W1, b1r)


def _logits_body(h_ref, w2t_ref, b2_ref, logit_ref, lse_ref, m_ref, s_ref):
    j = pl.program_id(0)

    @pl.when(j == 0)
    def _():
        m_ref[0] = -jnp.inf
        s_ref[0] = 0.0

    tile = lax.dot_general(
        h_ref[...], w2t_ref[...], (((1,), (1,)), ((), ())),
        preferred_element_type=jnp.float32) + b2_ref[...]
    col = j * _BV + lax.broadcasted_iota(jnp.int32, (1, _BV), 1)
    valid = col < CARDS
    tmax = jnp.max(jnp.where(valid, tile, -jnp.inf))
    m_old = m_ref[0]
    m_new = jnp.maximum(m_old, tmax)
    s_ref[0] = (s_ref[0] * jnp.exp(m_old - m_new)
                + jnp.sum(jnp.where(valid, jnp.exp(tile - m_new), 0.0)))
    m_ref[0] = m_new
    logit_ref[...] = tile
    lse_ref[0, 0] = m_new + jnp.log(s_ref[0])


def _logits_layer(h, W2T, b2r):
    return pl.pallas_call(
        _logits_body,
        grid=(_GB,),
        in_specs=[
            pl.BlockSpec((1, HID), lambda j: (0, 0)),
            pl.BlockSpec((_BV, HID), lambda j: (j, 0)),
            pl.BlockSpec((1, _BV), lambda j: (0, j)),
        ],
        out_specs=[
            pl.BlockSpec((1, _BV), lambda j: (0, j)),
            pl.BlockSpec(memory_space=pltpu.SMEM),
        ],
        out_shape=[
            jax.ShapeDtypeStruct((1, CARDS), jnp.float32),
            jax.ShapeDtypeStruct((1, 1), jnp.float32),
        ],
        scratch_shapes=[
            pltpu.SMEM((1,), jnp.float32),
            pltpu.SMEM((1,), jnp.float32),
        ],
    )(h, W2T, b2r)


def _logsub_body(logit_ref, lse_ref, out_ref):
    out_ref[...] = logit_ref[...] - lse_ref[0, 0]


def _logsub(logits, lse):
    return pl.pallas_call(
        _logsub_body,
        in_specs=[
            pl.BlockSpec((1, CARDS), lambda: (0, 0)),
            pl.BlockSpec(memory_space=pltpu.SMEM),
        ],
        out_specs=pl.BlockSpec((1, CARDS), lambda: (0, 0)),
        out_shape=jax.ShapeDtypeStruct((1, CARDS), jnp.float32),
    )(logits, lse)


def kernel(inputs, emb_table, W1, b1, W2, b2):
    idx = inputs.astype(jnp.int32)
    idx_pad = jnp.zeros((_CTX_PAD,), jnp.int32).at[:CTX].set(idx)
    embeds = _sc_gather(emb_table, idx_pad)
    x = embeds[:CTX].reshape(1, CTX * EMB_D)
    b1r = b1.reshape(1, HID)
    b2r = b2.reshape(1, CARDS)
    h = _h_layer(x, W1, b1r)
    logits, lse = _logits_layer(h, W2.T, b2r)
    return _logsub(logits, lse)


# speedup vs baseline: 1.0019x; 1.0019x over previous
import functools

import jax
import jax.numpy as jnp
from jax import lax
from jax.experimental import pallas as pl
from jax.experimental.pallas import tpu as pltpu
from jax.experimental.pallas import tpu_sc as plsc

CARDS = 100000
EMB_D = 128
CTX = 200
HID = 128

_NC = 2
_NS = 16
_NW = _NC * _NS
_CTX_PAD = 256
_BPW = _CTX_PAD // _NW

_KS = 8
_BK = CTX * EMB_D // _KS

_BV = 8192
_GB = (CARDS + _BV - 1) // _BV

_STEPS = _KS + _GB


def _sc_gather(table, idx_pad):
    mesh = plsc.VectorSubcoreMesh(core_axis_name="c", subcore_axis_name="s")

    @functools.partial(
        pl.kernel,
        mesh=mesh,
        out_type=jax.ShapeDtypeStruct((_CTX_PAD, EMB_D), jnp.float32),
        scratch_types=[
            pltpu.VMEM((_BPW,), jnp.int32),
            pltpu.VMEM((_BPW, EMB_D), jnp.float32),
            pltpu.SemaphoreType.DMA,
        ],
    )
    def k(table_hbm, idx_hbm, out_hbm, idx_v, rows_v, sem):
        wid = lax.axis_index("s") * _NC + lax.axis_index("c")
        base = wid * _BPW
        pltpu.sync_copy(idx_hbm.at[pl.ds(base, _BPW)], idx_v)
        pltpu.async_copy(table_hbm.at[idx_v], rows_v, sem).wait()
        pltpu.sync_copy(rows_v, out_hbm.at[pl.ds(base, _BPW)])

    return k(table, idx_pad)


def _fused_body(x_ref, w1_ref, b1_ref, w2t_ref, b2_ref,
                logit_ref, lse_ref, h_ref, m_ref, s_ref):
    j = pl.program_id(0)

    @pl.when(j == 0)
    def _():
        h_ref[...] = jnp.zeros_like(h_ref)
        m_ref[0] = -jnp.inf
        s_ref[0] = 0.0

    @pl.when(j < _KS)
    def _():
        acc = h_ref[...] + jnp.dot(x_ref[...], w1_ref[...],
                                   preferred_element_type=jnp.float32)
        h_ref[...] = jnp.where(
            j == _KS - 1, jnp.maximum(acc + b1_ref[...], 0.0), acc)

    @pl.when(j >= _KS)
    def _():
        jv = j - _KS
        tile = lax.dot_general(
            h_ref[...], w2t_ref[...], (((1,), (1,)), ((), ())),
            preferred_element_type=jnp.float32) + b2_ref[...]
        col = jv * _BV + lax.broadcasted_iota(jnp.int32, (1, _BV), 1)
        valid = col < CARDS
        tmax = jnp.max(jnp.where(valid, tile, -jnp.inf))
        m_old = m_ref[0]
        m_new = jnp.maximum(m_old, tmax)
        s_ref[0] = (s_ref[0] * jnp.exp(m_old - m_new)
                    + jnp.sum(jnp.where(valid, jnp.exp(tile - m_new), 0.0)))
        m_ref[0] = m_new
        logit_ref[...] = tile
        lse_ref[0, 0] = m_new + jnp.log(s_ref[0])


def _fused_layer(x, W1, b1r, W2T, b2r):
    kclip = lambda j: jnp.minimum(j, _KS - 1)
    vclip = lambda j: jnp.clip(j - _KS, 0, _GB - 1)
    return pl.pallas_call(
        _fused_body,
        grid=(_STEPS,),
        in_specs=[
            pl.BlockSpec((1, _BK), lambda j: (0, kclip(j))),
            pl.BlockSpec((_BK, HID), lambda j: (kclip(j), 0)),
            pl.BlockSpec((1, HID), lambda j: (0, 0)),
            pl.BlockSpec((_BV, HID), lambda j: (vclip(j), 0)),
            pl.BlockSpec((1, _BV), lambda j: (0, vclip(j))),
        ],
        out_specs=[
            pl.BlockSpec((1, _BV), lambda j: (0, vclip(j))),
            pl.BlockSpec(memory_space=pltpu.SMEM),
        ],
        out_shape=[
            jax.ShapeDtypeStruct((1, CARDS), jnp.float32),
            jax.ShapeDtypeStruct((1, 1), jnp.float32),
        ],
        scratch_shapes=[
            pltpu.VMEM((1, HID), jnp.float32),
            pltpu.SMEM((1,), jnp.float32),
            pltpu.SMEM((1,), jnp.float32),
        ],
    )(x, W1, b1r, W2T, b2r)


def _logsub_body(logit_ref, lse_ref, out_ref):
    out_ref[...] = logit_ref[...] - lse_ref[0, 0]


def _logsub(logits, lse):
    return pl.pallas_call(
        _logsub_body,
        in_specs=[
            pl.BlockSpec((1, CARDS), lambda: (0, 0)),
            pl.BlockSpec(memory_space=pltpu.SMEM),
        ],
        out_specs=pl.BlockSpec((1, CARDS), lambda: (0, 0)),
        out_shape=jax.ShapeDtypeStruct((1, CARDS), jnp.float32),
    )(logits, lse)


def kernel(inputs, emb_table, W1, b1, W2, b2):
    idx = inputs.astype(jnp.int32)
    idx_pad = jnp.zeros((_CTX_PAD,), jnp.int32).at[:CTX].set(idx)
    embeds = _sc_gather(emb_table, idx_pad)
    x = embeds[:CTX].reshape(1, CTX * EMB_D)
    b1r = b1.reshape(1, HID)
    b2r = b2.reshape(1, CARDS)
    logits, lse = _fused_layer(x, W1, b1r, W2.T, b2r)
    return _logsub(logits, lse)


# speedup vs baseline: 1.1107x; 1.1086x over previous
import functools

import jax
import jax.numpy as jnp
from jax import lax
from jax.experimental import pallas as pl
from jax.experimental.pallas import tpu as pltpu
from jax.experimental.pallas import tpu_sc as plsc

CARDS = 100000
EMB_D = 128
CTX = 200
HID = 128

_NC = 2
_NS = 16
_NW = _NC * _NS
_CTX_PAD = 256
_BPW = _CTX_PAD // _NW

_KS = 4
_BK = CTX * EMB_D // _KS

_BV = 16384
_GB = (CARDS + _BV - 1) // _BV

_STEPS = _KS + 2 * _GB


def _sc_gather(table, idx_pad):
    mesh = plsc.VectorSubcoreMesh(core_axis_name="c", subcore_axis_name="s")

    @functools.partial(
        pl.kernel,
        mesh=mesh,
        out_type=jax.ShapeDtypeStruct((_CTX_PAD, EMB_D), jnp.float32),
        scratch_types=[
            pltpu.VMEM((_BPW,), jnp.int32),
            pltpu.VMEM((_BPW, EMB_D), jnp.float32),
            pltpu.SemaphoreType.DMA,
        ],
    )
    def k(table_hbm, idx_hbm, out_hbm, idx_v, rows_v, sem):
        wid = lax.axis_index("s") * _NC + lax.axis_index("c")
        base = wid * _BPW
        pltpu.sync_copy(idx_hbm.at[pl.ds(base, _BPW)], idx_v)
        pltpu.async_copy(table_hbm.at[idx_v], rows_v, sem).wait()
        pltpu.sync_copy(rows_v, out_hbm.at[pl.ds(base, _BPW)])

    return k(table, idx_pad)


def _fused_body(x_ref, w1_ref, b1_ref, w2t_ref, b2_ref,
                out_ref, h_ref, tiles_ref, m_ref, s_ref):
    j = pl.program_id(0)

    @pl.when(j == 0)
    def _():
        h_ref[...] = jnp.zeros_like(h_ref)
        m_ref[0] = -jnp.inf
        s_ref[0] = 0.0

    @pl.when(j < _KS)
    def _():
        acc = h_ref[...] + jnp.dot(x_ref[...], w1_ref[...],
                                   preferred_element_type=jnp.float32)
        h_ref[...] = jnp.where(
            j == _KS - 1, jnp.maximum(acc + b1_ref[...], 0.0), acc)

    @pl.when(jnp.logical_and(j >= _KS, j < _KS + _GB))
    def _():
        jv = j - _KS
        tile = lax.dot_general(
            h_ref[...], w2t_ref[...], (((1,), (1,)), ((), ())),
            preferred_element_type=jnp.float32) + b2_ref[...].reshape(1, _BV)
        col = jv * _BV + lax.broadcasted_iota(jnp.int32, (1, _BV), 1)
        valid = col < CARDS
        tmax = jnp.max(jnp.where(valid, tile, -jnp.inf))
        m_old = m_ref[0]
        m_new = jnp.maximum(m_old, tmax)
        s_ref[0] = (s_ref[0] * jnp.exp(m_old - m_new)
                    + jnp.sum(jnp.where(valid, jnp.exp(tile - m_new), 0.0)))
        m_ref[0] = m_new
        tiles_ref[jv] = tile

    @pl.when(j >= _KS + _GB)
    def _():
        jw = j - _KS - _GB
        out_ref[...] = tiles_ref[jw] - (m_ref[0] + jnp.log(s_ref[0]))


def _fused_layer(x, W1, b1r, W2T, b2):
    kclip = lambda j: jnp.minimum(j, _KS - 1)
    vclip = lambda j: jnp.clip(j - _KS, 0, _GB - 1)
    wclip = lambda j: jnp.clip(j - _KS - _GB, 0, _GB - 1)
    return pl.pallas_call(
        _fused_body,
        grid=(_STEPS,),
        in_specs=[
            pl.BlockSpec((1, _BK), lambda j: (0, kclip(j))),
            pl.BlockSpec((_BK, HID), lambda j: (kclip(j), 0)),
            pl.BlockSpec((1, HID), lambda j: (0, 0)),
            pl.BlockSpec((_BV, HID), lambda j: (vclip(j), 0)),
            pl.BlockSpec((_BV,), lambda j: (vclip(j),)),
        ],
        out_specs=pl.BlockSpec((1, _BV), lambda j: (0, wclip(j))),
        out_shape=jax.ShapeDtypeStruct((1, CARDS), jnp.float32),
        scratch_shapes=[
            pltpu.VMEM((1, HID), jnp.float32),
            pltpu.VMEM((_GB, 1, _BV), jnp.float32),
            pltpu.SMEM((1,), jnp.float32),
            pltpu.SMEM((1,), jnp.float32),
        ],
    )(x, W1, b1r, W2T, b2)


def kernel(inputs, emb_table, W1, b1, W2, b2):
    idx = inputs.astype(jnp.int32)
    idx_pad = jnp.zeros((_CTX_PAD,), jnp.int32).at[:CTX].set(idx)
    embeds = _sc_gather(emb_table, idx_pad)
    x = embeds[:CTX].reshape(1, CTX * EMB_D)
    b1r = b1.reshape(1, HID)
    return _fused_layer(x, W1, b1r, W2.T, b2)


# speedup vs baseline: 1.1769x; 1.0596x over previous
import functools

import jax
import jax.numpy as jnp
from jax import lax
from jax.experimental import pallas as pl
from jax.experimental.pallas import tpu as pltpu
from jax.experimental.pallas import tpu_sc as plsc

CARDS = 100000
EMB_D = 128
CTX = 200
HID = 128

_NC = 2
_NS = 16
_NW = _NC * _NS
_CTX_PAD = 256
_BPW = _CTX_PAD // _NW

_KS = 4
_BK = CTX * EMB_D // _KS

_BV = 25600
_GB = (CARDS + _BV - 1) // _BV

_STEPS = _KS + 2 * _GB


def _sc_gather(table, idx_pad):
    mesh = plsc.VectorSubcoreMesh(core_axis_name="c", subcore_axis_name="s")

    @functools.partial(
        pl.kernel,
        mesh=mesh,
        out_type=jax.ShapeDtypeStruct((_CTX_PAD, EMB_D), jnp.float32),
        scratch_types=[
            pltpu.VMEM((_BPW,), jnp.int32),
            pltpu.VMEM((_BPW, EMB_D), jnp.float32),
            pltpu.SemaphoreType.DMA,
        ],
    )
    def k(table_hbm, idx_hbm, out_hbm, idx_v, rows_v, sem):
        wid = lax.axis_index("s") * _NC + lax.axis_index("c")
        base = wid * _BPW
        pltpu.sync_copy(idx_hbm.at[pl.ds(base, _BPW)], idx_v)
        pltpu.async_copy(table_hbm.at[idx_v], rows_v, sem).wait()
        pltpu.sync_copy(rows_v, out_hbm.at[pl.ds(base, _BPW)])

    return k(table, idx_pad)


def _fused_body(x_ref, w1_ref, b1_ref, w2t_ref, b2_ref,
                out_ref, h_ref, tiles_ref, m_ref, s_ref):
    j = pl.program_id(0)

    @pl.when(j == 0)
    def _():
        h_ref[...] = jnp.zeros_like(h_ref)
        m_ref[0] = -jnp.inf
        s_ref[0] = 0.0

    @pl.when(j < _KS)
    def _():
        acc = h_ref[...] + jnp.dot(x_ref[...], w1_ref[...],
                                   preferred_element_type=jnp.float32)
        h_ref[...] = jnp.where(
            j == _KS - 1, jnp.maximum(acc + b1_ref[...], 0.0), acc)

    @pl.when(jnp.logical_and(j >= _KS, j < _KS + _GB))
    def _():
        jv = j - _KS
        tile = lax.dot_general(
            h_ref[...], w2t_ref[...], (((1,), (1,)), ((), ())),
            preferred_element_type=jnp.float32) + b2_ref[...].reshape(1, _BV)
        col = jv * _BV + lax.broadcasted_iota(jnp.int32, (1, _BV), 1)
        valid = col < CARDS
        tmax = jnp.max(jnp.where(valid, tile, -jnp.inf))
        m_old = m_ref[0]
        m_new = jnp.maximum(m_old, tmax)
        s_ref[0] = (s_ref[0] * jnp.exp(m_old - m_new)
                    + jnp.sum(jnp.where(valid, jnp.exp(tile - m_new), 0.0)))
        m_ref[0] = m_new
        tiles_ref[jv] = tile

    @pl.when(j >= _KS + _GB)
    def _():
        jw = j - _KS - _GB
        out_ref[...] = tiles_ref[jw] - (m_ref[0] + jnp.log(s_ref[0]))


def _fused_layer(x, W1, b1r, W2T, b2):
    kclip = lambda j: jnp.minimum(j, _KS - 1)
    vclip = lambda j: jnp.clip(j - _KS, 0, _GB - 1)
    wclip = lambda j: jnp.clip(j - _KS - _GB, 0, _GB - 1)
    return pl.pallas_call(
        _fused_body,
        grid=(_STEPS,),
        in_specs=[
            pl.BlockSpec((1, _BK), lambda j: (0, kclip(j))),
            pl.BlockSpec((_BK, HID), lambda j: (kclip(j), 0)),
            pl.BlockSpec((1, HID), lambda j: (0, 0)),
            pl.BlockSpec((_BV, HID), lambda j: (vclip(j), 0)),
            pl.BlockSpec((_BV,), lambda j: (vclip(j),)),
        ],
        out_specs=pl.BlockSpec((1, _BV), lambda j: (0, wclip(j))),
        out_shape=jax.ShapeDtypeStruct((1, CARDS), jnp.float32),
        scratch_shapes=[
            pltpu.VMEM((1, HID), jnp.float32),
            pltpu.VMEM((_GB, 1, _BV), jnp.float32),
            pltpu.SMEM((1,), jnp.float32),
            pltpu.SMEM((1,), jnp.float32),
        ],
    )(x, W1, b1r, W2T, b2)


def kernel(inputs, emb_table, W1, b1, W2, b2):
    idx = inputs.astype(jnp.int32)
    idx_pad = jnp.zeros((_CTX_PAD,), jnp.int32).at[:CTX].set(idx)
    embeds = _sc_gather(emb_table, idx_pad)
    x = embeds[:CTX].reshape(1, CTX * EMB_D)
    b1r = b1.reshape(1, HID)
    return _fused_layer(x, W1, b1r, W2.T, b2)
